# baseline (speedup 1.0000x reference)
import functools

import jax
import jax.numpy as jnp
from jax.experimental import pallas as pl
from jax.experimental.pallas import tpu as pltpu

HIGHEST = jax.lax.Precision.HIGHEST


def _dsa_kernel(x_ref, bnd_ref, w1t_ref, b1_ref, w2t_ref, b2_ref,
                wq_ref, bq_ref, wk_ref, bk_ref, wv_ref, bv_ref,
                wo_ref, bo_ref, lng_ref, lnb_ref,
                out_ref, imp_ref,
                sel_ref, rank_ref, xs_ref, q_ref, k_ref, v_ref, ctx_ref,
                *, N, K, heads, hd):
    f32 = jnp.float32
    C = x_ref.shape[2]
    scale = f32(hd) ** -0.5
    CH = 512

    x = x_ref[0]
    h1 = jax.lax.dot_general(x, w1t_ref[...], (((1,), (0,)), ((), ())),
                             precision=HIGHEST) + b1_ref[...]
    h1 = h1 * f32(0.5) * (f32(1.0) + jax.lax.erf(h1 * f32(0.7071067811865476)))
    logit = jax.lax.dot_general(h1, w2t_ref[...], (((1,), (0,)), ((), ())),
                                precision=HIGHEST) + b2_ref[...]
    imp = jax.nn.sigmoid(logit) + f32(0.5) * bnd_ref[0]
    imp_ref[0] = imp

    bits = jax.lax.bitcast_convert_type(imp, jnp.int32)

    def cnt_ge(thr):
        return jnp.sum((bits >= thr).astype(jnp.int32))

    def t_body(i, t):
        cand = t | (jnp.int32(1) << (jnp.int32(30) - i))
        return jnp.where(cnt_ge(cand) >= K, cand, t)

    t = jax.lax.fori_loop(0, 31, t_body, jnp.int32(0))

    cnt_gt = cnt_ge(t + 1)
    need = K - cnt_gt

    def key_of():
        idx_col = jax.lax.broadcasted_iota(jnp.int32, (N, 1), 0)
        return jnp.where(bits == t, jnp.int32(N - 1) - idx_col, jnp.int32(-1))

    def th_body(i, th):
        cand = th | (jnp.int32(1) << (jnp.int32(11) - i))
        cnt = jnp.sum((key_of() >= cand).astype(jnp.int32))
        return jnp.where(cnt >= need, cand, th)

    th2 = jax.lax.fori_loop(0, 12, th_body, jnp.int32(0))
    sel_ref[...] = ((bits > t) | (key_of() >= th2)).astype(f32)

    lane_i = jax.lax.broadcasted_iota(jnp.int32, (128, 128), 0)
    lane_j = jax.lax.broadcasted_iota(jnp.int32, (128, 128), 1)
    Lstrict = (lane_i < lane_j).astype(f32)

    def rank_body(c, off):
        s_c = sel_ref[pl.ds(c * 128, 128), :]
        p_c = jax.lax.dot_general(Lstrict, s_c, (((0,), (0,)), ((), ())),
                                  precision=HIGHEST)
        rank_ref[pl.ds(c * 128, 128), :] = p_c + off
        return off + jnp.sum(s_c)

    jax.lax.fori_loop(0, N // 128, rank_body, f32(0.0))

    def gt_chunk(c):
        rho = jax.lax.broadcasted_iota(jnp.int32, (CH, K), 1)
        rk = rank_ref[pl.ds(c * CH, CH), :].astype(jnp.int32)
        sl = sel_ref[pl.ds(c * CH, CH), :]
        return (rho == rk).astype(f32) * sl

    xs_ref[...] = jnp.zeros((K, C), dtype=f32)

    def gather_body(c, carry):
        xw = x_ref[0, pl.ds(c * CH, CH), :]
        xs_ref[...] += jax.lax.dot_general(
            gt_chunk(c), xw, (((0,), (0,)), ((), ())), precision=HIGHEST)
        return carry

    jax.lax.fori_loop(0, N // CH, gather_body, 0)

    def lin(a, w_ref, b_ref):
        return jax.lax.dot_general(a, w_ref[...], (((1,), (1,)), ((), ())),
                                   precision=HIGHEST) + b_ref[...]

    xs = xs_ref[...]
    q_ref[...] = lin(xs, wq_ref, bq_ref)
    k_ref[...] = lin(xs, wk_ref, bk_ref)
    v_ref[...] = lin(xs, wv_ref, bv_ref)
    ctx_ref[...] = jnp.zeros((K, C), dtype=f32)

    col = jax.lax.broadcasted_iota(jnp.int32, (1, C), 1)

    def head_body(h, carry):
        hm = ((col >= h * hd) & (col < (h + 1) * hd)).astype(f32)
        s_h = jax.lax.dot_general(q_ref[...] * hm, k_ref[...],
                                  (((1,), (1,)), ((), ())),
                                  precision=HIGHEST) * scale
        m = jnp.max(s_h, axis=1, keepdims=True)
        e = jnp.exp(s_h - m)
        p = e / jnp.sum(e, axis=1, keepdims=True)
        ctx_ref[...] += jax.lax.dot_general(p, v_ref[...] * hm,
                                            (((1,), (0,)), ((), ())),
                                            precision=HIGHEST)
        return carry

    jax.lax.fori_loop(0, heads, head_body, 0)

    y = lin(ctx_ref[...], wo_ref, bo_ref) + xs
    mu = jnp.mean(y, axis=1, keepdims=True)
    var = jnp.mean((y - mu) ** 2, axis=1, keepdims=True)
    enh = (y - mu) * jax.lax.rsqrt(var + f32(1e-5)) * lng_ref[...] + lnb_ref[...]
    q_ref[...] = enh

    def scatter_body(c, carry):
        scat_c = jax.lax.dot_general(gt_chunk(c), q_ref[...],
                                     (((1,), (0,)), ((), ())),
                                     precision=HIGHEST)
        sl = sel_ref[pl.ds(c * CH, CH), :]
        xw = x_ref[0, pl.ds(c * CH, CH), :]
        out_ref[0, pl.ds(c * CH, CH), :] = jnp.where(sl > f32(0.5), scat_c, xw)
        return carry

    jax.lax.fori_loop(0, N // CH, scatter_body, 0)


def kernel(x, boundary_map, w_imp1, b_imp1, w_imp2, b_imp2,
           Wq, bq, Wk, bk, Wv, bv, Wo, bo, ln_g, ln_b):
    B, C, H, W = x.shape
    N = H * W
    K = max(int(N * 0.25), 1)
    heads = 8
    hd = C // heads

    x_flat = jnp.transpose(x.reshape(B, C, N), (0, 2, 1))
    bnd = boundary_map.reshape(B, N, 1)

    full = lambda s: pl.BlockSpec(s, lambda b: (0,) * len(s))
    out_flat, imp = pl.pallas_call(
        functools.partial(_dsa_kernel, N=N, K=K, heads=heads, hd=hd),
        grid=(B,),
        in_specs=[
            pl.BlockSpec((1, N, C), lambda b: (b, 0, 0)),
            pl.BlockSpec((1, N, 1), lambda b: (b, 0, 0)),
            full((C, C // 4)), full((1, C // 4)),
            full((C // 4, 1)), full((1, 1)),
            full((C, C)), full((1, C)),
            full((C, C)), full((1, C)),
            full((C, C)), full((1, C)),
            full((C, C)), full((1, C)),
            full((1, C)), full((1, C)),
        ],
        out_specs=[
            pl.BlockSpec((1, N, C), lambda b: (b, 0, 0)),
            pl.BlockSpec((1, N, 1), lambda b: (b, 0, 0)),
        ],
        out_shape=[
            jax.ShapeDtypeStruct((B, N, C), jnp.float32),
            jax.ShapeDtypeStruct((B, N, 1), jnp.float32),
        ],
        scratch_shapes=[
            pltpu.VMEM((N, 1), jnp.float32),
            pltpu.VMEM((N, 1), jnp.float32),
            pltpu.VMEM((K, C), jnp.float32),
            pltpu.VMEM((K, C), jnp.float32),
            pltpu.VMEM((K, C), jnp.float32),
            pltpu.VMEM((K, C), jnp.float32),
            pltpu.VMEM((K, C), jnp.float32),
        ],
    )(x_flat, bnd,
      w_imp1.T, b_imp1.reshape(1, -1), w_imp2.T, b_imp2.reshape(1, 1),
      Wq, bq.reshape(1, -1), Wk, bk.reshape(1, -1), Wv, bv.reshape(1, -1),
      Wo, bo.reshape(1, -1), ln_g.reshape(1, -1), ln_b.reshape(1, -1))

    out = jnp.transpose(out_flat.reshape(B, H, W, C), (0, 3, 1, 2))
    importance = imp.reshape(B, 1, H, W)
    return (out, importance)


# speedup vs baseline: 1.4512x; 1.4512x over previous
import functools

import jax
import jax.numpy as jnp
from jax.experimental import pallas as pl
from jax.experimental.pallas import tpu as pltpu

HIGHEST = jax.lax.Precision.HIGHEST


def _dsa_kernel(x_ref, bnd_ref, w1t_ref, b1_ref, w2t_ref, b2_ref,
                wq_ref, bq_ref, wk_ref, bk_ref, wv_ref, bv_ref,
                wo_ref, bo_ref, lng_ref, lnb_ref,
                out_ref, imp_ref,
                sel_ref, rank_ref, xs_ref, q_ref, k_ref, v_ref, ctx_ref,
                *, N, K, heads, hd):
    f32 = jnp.float32
    C = x_ref.shape[2]
    scale = f32(hd) ** -0.5
    CH = 512

    x = x_ref[0]
    h1 = jax.lax.dot_general(x, w1t_ref[...], (((1,), (0,)), ((), ())),
                             precision=HIGHEST) + b1_ref[...]
    h1 = h1 * f32(0.5) * (f32(1.0) + jax.lax.erf(h1 * f32(0.7071067811865476)))
    logit = jax.lax.dot_general(h1, w2t_ref[...], (((1,), (0,)), ((), ())),
                                precision=HIGHEST) + b2_ref[...]
    imp = jax.nn.sigmoid(logit) + f32(0.5) * bnd_ref[0]
    imp_ref[0] = imp

    bits = jax.lax.bitcast_convert_type(imp, jnp.int32)

    def cnt_ge(thr):
        return jnp.sum((bits >= thr).astype(jnp.int32))

    def t_body(i, t):
        cand = t | (jnp.int32(1) << (jnp.int32(30) - i))
        return jnp.where(cnt_ge(cand) >= K, cand, t)

    t = jax.lax.fori_loop(0, 31, t_body, jnp.int32(0))

    cnt_gt = cnt_ge(t + 1)
    need = K - cnt_gt

    def key_of():
        idx_col = jax.lax.broadcasted_iota(jnp.int32, (N, 1), 0)
        return jnp.where(bits == t, jnp.int32(N - 1) - idx_col, jnp.int32(-1))

    def th_body(i, th):
        cand = th | (jnp.int32(1) << (jnp.int32(11) - i))
        cnt = jnp.sum((key_of() >= cand).astype(jnp.int32))
        return jnp.where(cnt >= need, cand, th)

    th2 = jax.lax.fori_loop(0, 12, th_body, jnp.int32(0))
    sel_ref[...] = ((bits > t) | (key_of() >= th2)).astype(f32)

    lane_i = jax.lax.broadcasted_iota(jnp.int32, (128, 128), 0)
    lane_j = jax.lax.broadcasted_iota(jnp.int32, (128, 128), 1)
    Lstrict = (lane_i < lane_j).astype(f32)

    def rank_body(c, off):
        s_c = sel_ref[pl.ds(c * 128, 128), :]
        p_c = jax.lax.dot_general(Lstrict, s_c, (((0,), (0,)), ((), ())),
                                  precision=HIGHEST)
        rank_ref[pl.ds(c * 128, 128), :] = p_c + off
        return off + jnp.sum(s_c)

    jax.lax.fori_loop(0, N // 128, rank_body, f32(0.0))

    def gt_chunk(c):
        rho = jax.lax.broadcasted_iota(jnp.int32, (CH, K), 1)
        rk = rank_ref[pl.ds(c * CH, CH), :].astype(jnp.int32)
        sl = sel_ref[pl.ds(c * CH, CH), :]
        return (rho == rk).astype(f32) * sl

    xs_ref[...] = jnp.zeros((K, C), dtype=f32)

    def gather_body(c, carry):
        xw = x_ref[0, pl.ds(c * CH, CH), :]
        xs_ref[...] += jax.lax.dot_general(
            gt_chunk(c), xw, (((0,), (0,)), ((), ())), precision=HIGHEST)
        return carry

    jax.lax.fori_loop(0, N // CH, gather_body, 0)

    def lin(a, w_ref, b_ref):
        return jax.lax.dot_general(a, w_ref[...], (((1,), (1,)), ((), ())),
                                   precision=jax.lax.Precision.DEFAULT) + b_ref[...]

    xs = xs_ref[...]
    q_ref[...] = lin(xs, wq_ref, bq_ref)
    k_ref[...] = lin(xs, wk_ref, bk_ref)
    v_ref[...] = lin(xs, wv_ref, bv_ref)
    ctx_ref[...] = jnp.zeros((K, C), dtype=f32)

    col = jax.lax.broadcasted_iota(jnp.int32, (1, C), 1)

    def head_body(h, carry):
        hm = ((col >= h * hd) & (col < (h + 1) * hd)).astype(f32)
        s_h = jax.lax.dot_general(q_ref[...] * hm, k_ref[...],
                                  (((1,), (1,)), ((), ())),
                                  precision=jax.lax.Precision.DEFAULT) * scale
        m = jnp.max(s_h, axis=1, keepdims=True)
        e = jnp.exp(s_h - m)
        p = e / jnp.sum(e, axis=1, keepdims=True)
        ctx_ref[...] += jax.lax.dot_general(p, v_ref[...] * hm,
                                            (((1,), (0,)), ((), ())),
                                            precision=jax.lax.Precision.DEFAULT)
        return carry

    jax.lax.fori_loop(0, heads, head_body, 0)

    y = lin(ctx_ref[...], wo_ref, bo_ref) + xs
    mu = jnp.mean(y, axis=1, keepdims=True)
    var = jnp.mean((y - mu) ** 2, axis=1, keepdims=True)
    enh = (y - mu) * jax.lax.rsqrt(var + f32(1e-5)) * lng_ref[...] + lnb_ref[...]
    q_ref[...] = enh

    def scatter_body(c, carry):
        scat_c = jax.lax.dot_general(gt_chunk(c), q_ref[...],
                                     (((1,), (0,)), ((), ())),
                                     precision=HIGHEST)
        sl = sel_ref[pl.ds(c * CH, CH), :]
        xw = x_ref[0, pl.ds(c * CH, CH), :]
        out_ref[0, pl.ds(c * CH, CH), :] = jnp.where(sl > f32(0.5), scat_c, xw)
        return carry

    jax.lax.fori_loop(0, N // CH, scatter_body, 0)


def kernel(x, boundary_map, w_imp1, b_imp1, w_imp2, b_imp2,
           Wq, bq, Wk, bk, Wv, bv, Wo, bo, ln_g, ln_b):
    B, C, H, W = x.shape
    N = H * W
    K = max(int(N * 0.25), 1)
    heads = 8
    hd = C // heads

    x_flat = jnp.transpose(x.reshape(B, C, N), (0, 2, 1))
    bnd = boundary_map.reshape(B, N, 1)

    full = lambda s: pl.BlockSpec(s, lambda b: (0,) * len(s))
    out_flat, imp = pl.pallas_call(
        functools.partial(_dsa_kernel, N=N, K=K, heads=heads, hd=hd),
        grid=(B,),
        in_specs=[
            pl.BlockSpec((1, N, C), lambda b: (b, 0, 0)),
            pl.BlockSpec((1, N, 1), lambda b: (b, 0, 0)),
            full((C, C // 4)), full((1, C // 4)),
            full((C // 4, 1)), full((1, 1)),
            full((C, C)), full((1, C)),
            full((C, C)), full((1, C)),
            full((C, C)), full((1, C)),
            full((C, C)), full((1, C)),
            full((1, C)), full((1, C)),
        ],
        out_specs=[
            pl.BlockSpec((1, N, C), lambda b: (b, 0, 0)),
            pl.BlockSpec((1, N, 1), lambda b: (b, 0, 0)),
        ],
        out_shape=[
            jax.ShapeDtypeStruct((B, N, C), jnp.float32),
            jax.ShapeDtypeStruct((B, N, 1), jnp.float32),
        ],
        compiler_params=pltpu.CompilerParams(
            dimension_semantics=("parallel",)),
        scratch_shapes=[
            pltpu.VMEM((N, 1), jnp.float32),
            pltpu.VMEM((N, 1), jnp.float32),
            pltpu.VMEM((K, C), jnp.float32),
            pltpu.VMEM((K, C), jnp.float32),
            pltpu.VMEM((K, C), jnp.float32),
            pltpu.VMEM((K, C), jnp.float32),
            pltpu.VMEM((K, C), jnp.float32),
        ],
    )(x_flat, bnd,
      w_imp1.T, b_imp1.reshape(1, -1), w_imp2.T, b_imp2.reshape(1, 1),
      Wq, bq.reshape(1, -1), Wk, bk.reshape(1, -1), Wv, bv.reshape(1, -1),
      Wo, bo.reshape(1, -1), ln_g.reshape(1, -1), ln_b.reshape(1, -1))

    out = jnp.transpose(out_flat.reshape(B, H, W, C), (0, 3, 1, 2))
    importance = imp.reshape(B, 1, H, W)
    return (out, importance)


# speedup vs baseline: 4.0091x; 2.7626x over previous
import functools

import jax
import jax.numpy as jnp
from jax.experimental import pallas as pl
from jax.experimental.pallas import tpu as pltpu

HIGHEST = jax.lax.Precision.HIGHEST


def _mm(a, b, dims, prec=jax.lax.Precision.DEFAULT):
    return jax.lax.dot_general(a, b, (dims, ((), ())), precision=prec)


def _dsa_kernel(xt_ref, x_ref, bnd_ref, w1_ref, b1_ref, w2_ref, b2_ref,
                wq_ref, bq_ref, wk_ref, bk_ref, wv_ref, bv_ref,
                wo_ref, bo_ref, lng_ref, lnb_ref,
                out_ref, imp_ref,
                sel_ref, rank_ref, xs_ref, q_ref, k_ref, v_ref, ctx_ref,
                *, N, K, heads, hd):
    f32 = jnp.float32
    C = x_ref.shape[2]
    scale = f32(hd) ** -0.5

    h1t = _mm(w1_ref[...], xt_ref[0], ((1,), (0,)), HIGHEST) + b1_ref[...]
    h1t = h1t * f32(0.5) * (f32(1.0) + jax.lax.erf(h1t * f32(0.7071067811865476)))
    logit = _mm(w2_ref[...], h1t, ((1,), (0,)), HIGHEST) + b2_ref[...]
    imp = jax.nn.sigmoid(logit) + f32(0.5) * bnd_ref[0]
    imp_ref[0] = imp

    bits = jax.lax.bitcast_convert_type(imp, jnp.int32)

    def cnt_ge(thr):
        return jnp.sum((bits >= thr).astype(jnp.int32))

    def t_body(i, t):
        cand = t | (jnp.int32(1) << (jnp.int32(30) - i))
        return jnp.where(cnt_ge(cand) >= K, cand, t)

    t = jax.lax.fori_loop(0, 31, t_body, jnp.int32(0))
    need = K - cnt_ge(t + 1)

    idx_row = jax.lax.broadcasted_iota(jnp.int32, (1, N), 1)
    key = jnp.where(bits == t, jnp.int32(N - 1) - idx_row, jnp.int32(-1))

    def th_body(i, th):
        cand = th | (jnp.int32(1) << (jnp.int32(11) - i))
        cnt = jnp.sum((key >= cand).astype(jnp.int32))
        return jnp.where(cnt >= need, cand, th)

    th2 = jax.lax.fori_loop(0, 12, th_body, jnp.int32(0))
    sel_row = ((bits > t) | (key >= th2)).astype(f32)

    sel32 = sel_row.reshape(N // 128, 128)
    lane_i = jax.lax.broadcasted_iota(jnp.int32, (128, 128), 0)
    lane_j = jax.lax.broadcasted_iota(jnp.int32, (128, 128), 1)
    Ustrict = (lane_i < lane_j).astype(f32)
    row_i = jax.lax.broadcasted_iota(jnp.int32, (N // 128, N // 128), 0)
    row_j = jax.lax.broadcasted_iota(jnp.int32, (N // 128, N // 128), 1)
    Lstrict = (row_j < row_i).astype(f32)
    prefix_in = _mm(sel32, Ustrict, ((1,), (0,)))
    rowsum = jnp.sum(sel32, axis=1, keepdims=True)
    offs = _mm(Lstrict, rowsum, ((1,), (0,)))
    rank32 = prefix_in + offs
    rank_row = rank32.reshape(1, N)
    sel_ref[...] = jnp.swapaxes(sel_row, 0, 1)
    rank_ref[...] = jnp.swapaxes(rank_row, 0, 1)

    GCH = 256
    rho0 = jax.lax.broadcasted_iota(jnp.int32, (GCH, N), 0)
    rank_i = rank_row.astype(jnp.int32)

    def gather_body(c, carry):
        G = ((rho0 == rank_i - c * GCH) & (sel_row > f32(0.5))).astype(f32)
        xs_ref[pl.ds(c * GCH, GCH), :] = _mm(G, x_ref[0], ((1,), (0,)))
        return carry

    jax.lax.fori_loop(0, K // GCH, gather_body, 0)

    def lin(a, w_ref, b_ref):
        return _mm(a, w_ref[...], ((1,), (1,))) + b_ref[...]

    xs = xs_ref[...]
    q_ref[...] = lin(xs, wq_ref, bq_ref)
    k_ref[...] = lin(xs, wk_ref, bk_ref)
    v_ref[...] = lin(xs, wv_ref, bv_ref)
    ctx_ref[...] = jnp.zeros((K, C), dtype=f32)

    col = jax.lax.broadcasted_iota(jnp.int32, (1, C), 1)

    def head_body(h, carry):
        hm = ((col >= h * hd) & (col < (h + 1) * hd)).astype(f32)
        e = jnp.exp(_mm(q_ref[...] * hm, k_ref[...], ((1,), (1,))) * scale)
        ctx = _mm(e, v_ref[...] * hm, ((1,), (0,)))
        denom = jnp.sum(e, axis=1, keepdims=True)
        ctx_ref[...] += ctx / denom
        return carry

    jax.lax.fori_loop(0, heads, head_body, 0)

    y = lin(ctx_ref[...], wo_ref, bo_ref) + xs
    mu = jnp.mean(y, axis=1, keepdims=True)
    var = jnp.mean((y - mu) ** 2, axis=1, keepdims=True)
    enh = (y - mu) * jax.lax.rsqrt(var + f32(1e-5)) * lng_ref[...] + lnb_ref[...]
    q_ref[...] = enh

    SCH = 512
    rho1 = jax.lax.broadcasted_iota(jnp.int32, (SCH, K), 1)

    def scatter_body(c, carry):
        rk = rank_ref[pl.ds(c * SCH, SCH), :].astype(jnp.int32)
        sl = sel_ref[pl.ds(c * SCH, SCH), :]
        H = ((rho1 == rk) & (sl > f32(0.5))).astype(f32)
        scat = _mm(H, q_ref[...], ((1,), (0,)))
        xw = x_ref[0, pl.ds(c * SCH, SCH), :]
        out_ref[0, pl.ds(c * SCH, SCH), :] = jnp.where(sl > f32(0.5), scat, xw)
        return carry

    jax.lax.fori_loop(0, N // SCH, scatter_body, 0)


def kernel(x, boundary_map, w_imp1, b_imp1, w_imp2, b_imp2,
           Wq, bq, Wk, bk, Wv, bv, Wo, bo, ln_g, ln_b):
    B, C, H, W = x.shape
    N = H * W
    K = max(int(N * 0.25), 1)
    heads = 8
    hd = C // heads

    x_t = x.reshape(B, C, N)
    x_flat = jnp.transpose(x_t, (0, 2, 1))
    bnd = boundary_map.reshape(B, 1, N)

    full = lambda s: pl.BlockSpec(s, lambda b: (0,) * len(s))
    out_flat, imp = pl.pallas_call(
        functools.partial(_dsa_kernel, N=N, K=K, heads=heads, hd=hd),
        grid=(B,),
        in_specs=[
            pl.BlockSpec((1, C, N), lambda b: (b, 0, 0)),
            pl.BlockSpec((1, N, C), lambda b: (b, 0, 0)),
            pl.BlockSpec((1, 1, N), lambda b: (b, 0, 0)),
            full((C // 4, C)), full((C // 4, 1)),
            full((1, C // 4)), full((1, 1)),
            full((C, C)), full((1, C)),
            full((C, C)), full((1, C)),
            full((C, C)), full((1, C)),
            full((C, C)), full((1, C)),
            full((1, C)), full((1, C)),
        ],
        out_specs=[
            pl.BlockSpec((1, N, C), lambda b: (b, 0, 0)),
            pl.BlockSpec((1, 1, N), lambda b: (b, 0, 0)),
        ],
        out_shape=[
            jax.ShapeDtypeStruct((B, N, C), jnp.float32),
            jax.ShapeDtypeStruct((B, 1, N), jnp.float32),
        ],
        compiler_params=pltpu.CompilerParams(
            dimension_semantics=("parallel",)),
        scratch_shapes=[
            pltpu.VMEM((N, 1), jnp.float32),
            pltpu.VMEM((N, 1), jnp.float32),
            pltpu.VMEM((K, C), jnp.float32),
            pltpu.VMEM((K, C), jnp.float32),
            pltpu.VMEM((K, C), jnp.float32),
            pltpu.VMEM((K, C), jnp.float32),
            pltpu.VMEM((K, C), jnp.float32),
        ],
    )(x_t, x_flat, bnd,
      w_imp1, b_imp1.reshape(-1, 1), w_imp2, b_imp2.reshape(1, 1),
      Wq, bq.reshape(1, -1), Wk, bk.reshape(1, -1), Wv, bv.reshape(1, -1),
      Wo, bo.reshape(1, -1), ln_g.reshape(1, -1), ln_b.reshape(1, -1))

    out = jnp.transpose(out_flat.reshape(B, H, W, C), (0, 3, 1, 2))
    importance = imp.reshape(B, 1, H, W)
    return (out, importance)
